# Initial kernel scaffold; baseline (speedup 1.0000x reference)
import jax
import jax.numpy as jnp
from jax.experimental import pallas as pl


def kernel(subword_idx, offsets, table):
    raise NotImplementedError("write your pallas kernel here")


# speedup vs baseline: 26.2025x; 26.2025x over previous
import functools

import jax
import jax.numpy as jnp
from jax import lax
from jax.experimental import pallas as pl
from jax.experimental.pallas import tpu as pltpu
from jax.experimental.pallas import tpu_sc as plsc

LANES = 16
CHUNK = 128
NW = 32


def _make_sc_call(T, B, V, D):
    bags_per_w = B // NW
    nvec = D // LANES

    mesh = plsc.VectorSubcoreMesh(core_axis_name="c", subcore_axis_name="s")

    @functools.partial(
        pl.kernel,
        out_type=jax.ShapeDtypeStruct((B, D), jnp.float32),
        mesh=mesh,
        compiler_params=pltpu.CompilerParams(
            needs_layout_passes=False, use_tc_tiling_on_sc=False),
        scratch_types=[
            pltpu.VMEM((CHUNK,), jnp.int32),
            pltpu.VMEM((CHUNK, D), jnp.float32),
            pltpu.VMEM((bags_per_w + 2 * LANES,), jnp.int32),
            pltpu.VMEM((bags_per_w, D), jnp.float32),
            pltpu.SMEM((2,), jnp.int32),
        ],
    )
    def sc_embed(sub_hbm, offext_hbm, table_hbm, out_hbm,
                 idx_v, rows_v, off_v, outv, scal):
        cid = lax.axis_index("c")
        sid = lax.axis_index("s")
        wid = cid * 16 + sid
        b0 = wid * bags_per_w

        pltpu.sync_copy(
            offext_hbm.at[pl.ds(b0, bags_per_w + 2 * LANES)], off_v)

        ov = off_v[pl.ds(0, LANES)]
        p0 = ov[0]
        p0_al = (p0 >> 3) << 3

        def refill(chunk):
            start = pl.multiple_of(p0_al + chunk * CHUNK, 8)
            pltpu.sync_copy(sub_hbm.at[pl.ds(start, CHUNK)], idx_v)
            pltpu.sync_copy(table_hbm.at[idx_v], rows_v)

        refill(0)
        scal[0] = p0 - p0_al
        scal[1] = 0

        def bag_body(bl, _):
            vals = off_v[pl.ds(bl, LANES)]
            n = vals[1] - vals[0]

            def row_body(_, acc):
                @pl.when(scal[0] == CHUNK)
                def _():
                    refill(scal[1] + 1)
                    scal[1] = scal[1] + 1
                    scal[0] = 0
                p = scal[0]
                new = tuple(
                    acc[c] + rows_v[p, pl.ds(c * LANES, LANES)]
                    for c in range(nvec))
                scal[0] = p + 1
                return new

            zero = jnp.zeros((LANES,), jnp.float32)
            acc = lax.fori_loop(0, n, row_body, (zero,) * nvec)
            nv = jnp.maximum(
                jnp.broadcast_to(n, (LANES,)).astype(jnp.float32), 1.0)
            for c in range(nvec):
                outv[bl, pl.ds(c * LANES, LANES)] = acc[c] / nv
            return _

        lax.fori_loop(0, bags_per_w, bag_body, 0)
        pltpu.sync_copy(outv, out_hbm.at[pl.ds(b0, bags_per_w)])

    return sc_embed


def kernel(subword_idx, offsets, table):
    T = subword_idx.shape[0]
    B = offsets.shape[0]
    V, D = table.shape

    sub_pad = jnp.concatenate(
        [subword_idx, jnp.zeros((2 * CHUNK,), subword_idx.dtype)])
    offext = jnp.concatenate(
        [offsets, jnp.full((2 * LANES,), T, offsets.dtype)])

    return _make_sc_call(T, B, V, D)(sub_pad, offext, table)
